# baseline (speedup 1.0000x reference)
import functools

import jax
import jax.numpy as jnp
from jax import lax
from jax.experimental import pallas as pl
from jax.experimental.pallas import tpu as pltpu
from jax.experimental.pallas import tpu_sc as plsc

N = 10000
E = 320000
L = 5
POS = 32
H = 128
NIN = POS + 1
C = 40
G = 256
RESIDUAL_SCALE = 0.01

NTILES = 16
KCHUNK = 72
NSET = 5
CHUNKS_PT = 280
EPT = CHUNKS_PT * KCHUNK
EPAD = EPT * NTILES
NPAD = 10240
ROWS_PT = NPAD // NTILES

BLK = 2000


def _sc_agg_body(srcs_hbm, dst_hbm, table_hbm, zeros_hbm, out_hbm, agg_sh,
                 *bufs):
    c = lax.axis_index("c")
    s = lax.axis_index("s")
    srcs = bufs[0:NSET]
    dsts = bufs[NSET:2 * NSET]
    rows = bufs[2 * NSET:3 * NSET]
    gsems = bufs[3 * NSET:4 * NSET]
    ssems = bufs[4 * NSET:5 * NSET]
    isems = bufs[5 * NSET:6 * NSET]

    pltpu.sync_copy(zeros_hbm, rows[0])
    for j in range(ROWS_PT // KCHUNK):
        pltpu.sync_copy(
            rows[0], agg_sh.at[pl.ds(s * ROWS_PT + j * KCHUNK, KCHUNK)])
    rem = ROWS_PT - (ROWS_PT // KCHUNK) * KCHUNK
    if rem:
        pltpu.sync_copy(
            rows[0].at[pl.ds(0, rem)],
            agg_sh.at[pl.ds(s * ROWS_PT + ROWS_PT - rem, rem)])
    plsc.subcore_barrier()
    base = s * EPT

    def idx_load(t, q):
        off = base + t * KCHUNK
        pltpu.sync_copy(srcs_hbm.at[pl.ds(c * EPAD + off, KCHUNK)], srcs[q])
        pltpu.sync_copy(dst_hbm.at[pl.ds(off, KCHUNK)], dsts[q])

    def idx_issue(t, q):
        off = base + t * KCHUNK
        d1 = pltpu.async_copy(
            srcs_hbm.at[pl.ds(c * EPAD + off, KCHUNK)], srcs[q], isems[q])
        d2 = pltpu.async_copy(
            dst_hbm.at[pl.ds(off, KCHUNK)], dsts[q], isems[q])
        return d1, d2

    def idx_wait(t, q):
        off = base + t * KCHUNK
        pltpu.make_async_copy(
            srcs_hbm.at[pl.ds(c * EPAD + off, KCHUNK)], srcs[q],
            isems[q]).wait()
        pltpu.make_async_copy(
            dst_hbm.at[pl.ds(off, KCHUNK)], dsts[q], isems[q]).wait()

    def g_issue(q):
        pltpu.async_copy(table_hbm.at[srcs[q]], rows[q], gsems[q])

    def g_wait(q):
        pltpu.make_async_copy(table_hbm.at[srcs[q]], rows[q], gsems[q]).wait()

    def s_issue(q):
        pltpu.async_copy(rows[q], agg_sh.at[dsts[q]], ssems[q], add=True)

    def s_wait(q):
        pltpu.make_async_copy(rows[q], agg_sh.at[dsts[q]], ssems[q]).wait()

    def step(t, q, first=False):
        q2 = (q + 2) % NSET
        q3 = (q + 3) % NSET
        if not first:
            s_wait(q3)
        idx_issue(t + 3, q3)
        idx_wait(t + 2, q2)
        g_issue(q2)
        g_wait(q)
        s_issue(q)

    idx_load(0, 0)
    idx_load(1, 1)
    g_issue(0)
    idx_issue(2, 2)
    g_issue(1)
    step(0, 0, first=True)
    step(1, 1, first=True)
    step(2, 2)
    step(3, 3)
    step(4, 4)

    def five(p, carry):
        t = 5 * p
        for i in range(5):
            step(t + i, i)
        return carry

    lax.fori_loop(1, CHUNKS_PT // 5, five, 0)
    g_wait(0)
    g_wait(1)
    idx_wait(CHUNKS_PT + 2, (CHUNKS_PT + 2) % NSET)
    s_wait((CHUNKS_PT - 2) % NSET)
    s_wait((CHUNKS_PT - 1) % NSET)
    plsc.subcore_barrier()
    pltpu.sync_copy(agg_sh.at[pl.ds(s * ROWS_PT, ROWS_PT)],
                    out_hbm.at[pl.ds(c * NPAD + s * ROWS_PT, ROWS_PT)])


@functools.lru_cache(maxsize=1)
def _sc_agg_kernel():
    return pl.kernel(
        _sc_agg_body,
        out_type=jax.ShapeDtypeStruct((2 * NPAD, H), jnp.float32),
        mesh=plsc.VectorSubcoreMesh(
            core_axis_name="c", subcore_axis_name="s",
            num_cores=2, num_subcores=NTILES),
        scratch_types=(
            [pltpu.VMEM_SHARED((NPAD, H), jnp.float32)]
            + [pltpu.VMEM((KCHUNK,), jnp.int32) for _ in range(2 * NSET)]
            + [pltpu.VMEM((KCHUNK, H), jnp.float32) for _ in range(NSET)]
            + [pltpu.SemaphoreType.DMA for _ in range(3 * NSET)]
        ),
    )


def _agg_call(srcs, dstp, table, zeros_blk):
    out = _sc_agg_kernel()(srcs, dstp, table, zeros_blk)
    return out.reshape(2, NPAD, H)


def _prep_body(x_ref, xs0_ref, root_ref, cw_ref, cb_ref, aw_ref, ab_ref,
               tab_ref):
    i = pl.program_id(0)
    xb = x_ref[...]
    rows = lax.broadcasted_iota(jnp.int32, (BLK, G), 0) + i * BLK
    m = (rows == root_ref[...]).astype(jnp.float32)
    seed = jnp.minimum(jnp.sum(m, axis=1, keepdims=True), 1.0)
    h0 = jnp.concatenate(
        [xb, seed, jnp.zeros((BLK, H - NIN), jnp.float32)], axis=1)
    cond0 = jnp.dot(xs0_ref[...], cw_ref[...],
                    preferred_element_type=jnp.float32) + cb_ref[...]
    cfirst = jnp.dot(cond0, aw_ref[...],
                     preferred_element_type=jnp.float32) + ab_ref[...]
    tab_ref[0] = h0
    tab_ref[1] = h0 + cfirst


def _prep_call(x, xs0, root2d, cond_w, cond_b2d, aw, ab):
    full = lambda i: (0, 0)
    return pl.pallas_call(
        _prep_body,
        grid=(N // BLK,),
        in_specs=[
            pl.BlockSpec((BLK, POS), lambda i: (i, 0)),
            pl.BlockSpec((BLK, POS), lambda i: (i, 0)),
            pl.BlockSpec((1, G), full),
            pl.BlockSpec((POS, H), full),
            pl.BlockSpec((1, H), full),
            pl.BlockSpec((H, H), full),
            pl.BlockSpec((1, H), full),
        ],
        out_specs=pl.BlockSpec((2, BLK, H), lambda i: (0, i, 0)),
        out_shape=jax.ShapeDtypeStruct((2, N, H), jnp.float32),
    )(x, xs0, root2d, cond_w, cond_b2d, aw, ab)


def _layer_body(tab_ref, agg_ref, xsn_ref, wf_ref, bf_ref, wc_ref, bc_ref,
                zw_ref, zb_ref, cw_ref, cb_ref, acc_ref,
                tabn_ref, accn_ref):
    hf = jnp.maximum(
        jnp.dot(tab_ref[0] + agg_ref[0], wf_ref[...],
                preferred_element_type=jnp.float32) + bf_ref[...], 0.0)
    hc = jnp.maximum(
        jnp.dot(tab_ref[1] + agg_ref[1], wc_ref[...],
                preferred_element_type=jnp.float32) + bc_ref[...], 0.0)
    z = jnp.dot(hc, zw_ref[...], preferred_element_type=jnp.float32) \
        + zb_ref[...]
    hf_new = hf + RESIDUAL_SCALE * z
    condn = jnp.dot(xsn_ref[...], cw_ref[...],
                    preferred_element_type=jnp.float32) + cb_ref[...]
    tabn_ref[0] = hf_new
    tabn_ref[1] = hc + condn
    accn_ref[...] = acc_ref[...] + hf_new


def _layer_call(tab, agg, xsn, wf, bf, wc, bc, zw, zb, cond_w, cond_b2d, acc):
    full = lambda i: (0, 0)
    return pl.pallas_call(
        _layer_body,
        grid=(N // BLK,),
        in_specs=[
            pl.BlockSpec((2, BLK, H), lambda i: (0, i, 0)),
            pl.BlockSpec((2, BLK, H), lambda i: (0, i, 0)),
            pl.BlockSpec((BLK, POS), lambda i: (i, 0)),
            pl.BlockSpec((H, H), full),
            pl.BlockSpec((1, H), full),
            pl.BlockSpec((H, H), full),
            pl.BlockSpec((1, H), full),
            pl.BlockSpec((H, H), full),
            pl.BlockSpec((1, H), full),
            pl.BlockSpec((POS, H), full),
            pl.BlockSpec((1, H), full),
            pl.BlockSpec((BLK, H), lambda i: (i, 0)),
        ],
        out_specs=[
            pl.BlockSpec((2, BLK, H), lambda i: (0, i, 0)),
            pl.BlockSpec((BLK, H), lambda i: (i, 0)),
        ],
        out_shape=[
            jax.ShapeDtypeStruct((2, N, H), jnp.float32),
            jax.ShapeDtypeStruct((N, H), jnp.float32),
        ],
    )(tab, agg, xsn, wf, bf, wc, bc, zw, zb, cond_w, cond_b2d, acc)


def _pool_body(acc_ref, batch_ref, clsw_ref, clsb_ref, out_ref):
    oh = (lax.broadcasted_iota(jnp.int32, (G, N), 0)
          == batch_ref[...]).astype(jnp.float32)
    pooled = jnp.dot(oh, acc_ref[...], preferred_element_type=jnp.float32)
    cnt = jnp.sum(oh, axis=1, keepdims=True)
    pooled = pooled / jnp.maximum(cnt, 1.0)
    nrm = jnp.sqrt(jnp.sum(pooled * pooled, axis=1, keepdims=True))
    pooled = pooled / jnp.maximum(nrm, 1e-5)
    out_ref[...] = jnp.dot(pooled, clsw_ref[...],
                           preferred_element_type=jnp.float32) + clsb_ref[...]


def _pool_call(acc, batch2d, clsw_pad, clsb_pad):
    return pl.pallas_call(
        _pool_body,
        out_shape=jax.ShapeDtypeStruct((G, H), jnp.float32),
    )(acc, batch2d, clsw_pad, clsb_pad)


def kernel(x, x_sim_list, edge_index, batch, root_n_id, enc_w0, enc_b0,
           enc_w, enc_b, ctrl_w0, ctrl_b0, ctrl_w, ctrl_b, cond_w, cond_b,
           adapt_w, adapt_b, zero_w, zero_b, cls_w, cls_b):
    f32 = jnp.float32
    src = jnp.pad(edge_index[0], (0, EPAD - E))
    srcs = jnp.pad(jnp.concatenate([src, src + N]), (0, 3 * KCHUNK))
    dstp = jnp.pad(edge_index[1], (0, EPAD - E + 3 * KCHUNK),
                   constant_values=N)
    zeros_blk = jnp.zeros((KCHUNK, H), f32)

    wf0 = jnp.pad(enc_w0, ((0, H - NIN), (0, 0)))
    wc0 = jnp.pad(ctrl_w0, ((0, H - NIN), (0, 0)))
    aw = jnp.pad(adapt_w, ((0, 0), (0, H - NIN)))
    ab = jnp.pad(adapt_b, (0, H - NIN))[None]
    cond_b2d = cond_b[None]

    tab = _prep_call(x, x_sim_list[0], root_n_id[None].astype(jnp.int32),
                     cond_w, cond_b2d, aw, ab)
    acc = jnp.zeros((N, H), f32)
    for i in range(L):
        agg = _agg_call(srcs, dstp, tab.reshape(2 * N, H), zeros_blk)
        wf = wf0 if i == 0 else enc_w[i - 1]
        bf = (enc_b0 if i == 0 else enc_b[i - 1])[None]
        wc = wc0 if i == 0 else ctrl_w[i - 1]
        bc = (ctrl_b0 if i == 0 else ctrl_b[i - 1])[None]
        xsn = x_sim_list[(i + 1) % L]
        tab, acc = _layer_call(tab, agg, xsn, wf, bf, wc, bc,
                               zero_w[i], zero_b[i][None], cond_w, cond_b2d,
                               acc)
    out = _pool_call(acc, batch[None], jnp.pad(cls_w, ((0, 0), (0, H - C))),
                     jnp.pad(cls_b, (0, H - C))[None])
    return out[:, :C]


# speedup vs baseline: 1.3477x; 1.3477x over previous
import functools

import jax
import jax.numpy as jnp
from jax import lax
from jax.experimental import pallas as pl
from jax.experimental.pallas import tpu as pltpu
from jax.experimental.pallas import tpu_sc as plsc

N = 10000
E = 320000
L = 5
POS = 32
H = 128
NIN = POS + 1
C = 40
G = 256
RESIDUAL_SCALE = 0.01

NTILES = 16
KCHUNK = 88
NSET = 4
CHUNKS_PT = 228
EPT = CHUNKS_PT * KCHUNK
EPAD = EPT * NTILES
NPAD = 10240
ROWS_PT = NPAD // NTILES

BLK = 2000


def _sc_agg_body(srcs_hbm, dst_hbm, table_hbm, zeros_hbm, out_hbm, agg_sh,
                 *bufs):
    c = lax.axis_index("c")
    s = lax.axis_index("s")
    srcs = bufs[0:NSET]
    dsts = bufs[NSET:2 * NSET]
    rows = bufs[2 * NSET:3 * NSET]
    gsems = bufs[3 * NSET:4 * NSET]
    ssems = bufs[4 * NSET:5 * NSET]
    isems = bufs[5 * NSET:6 * NSET]

    pltpu.sync_copy(zeros_hbm, rows[0])
    for j in range(ROWS_PT // KCHUNK):
        pltpu.sync_copy(
            rows[0], agg_sh.at[pl.ds(s * ROWS_PT + j * KCHUNK, KCHUNK)])
    rem = ROWS_PT - (ROWS_PT // KCHUNK) * KCHUNK
    if rem:
        pltpu.sync_copy(
            rows[0].at[pl.ds(0, rem)],
            agg_sh.at[pl.ds(s * ROWS_PT + ROWS_PT - rem, rem)])
    plsc.subcore_barrier()
    base = s * EPT

    def idx_load(t, q):
        off = base + t * KCHUNK
        pltpu.sync_copy(srcs_hbm.at[pl.ds(c * EPAD + off, KCHUNK)], srcs[q])
        pltpu.sync_copy(dst_hbm.at[pl.ds(off, KCHUNK)], dsts[q])

    def idx_issue(t, q):
        off = base + t * KCHUNK
        d1 = pltpu.async_copy(
            srcs_hbm.at[pl.ds(c * EPAD + off, KCHUNK)], srcs[q], isems[q])
        d2 = pltpu.async_copy(
            dst_hbm.at[pl.ds(off, KCHUNK)], dsts[q], isems[q])
        return d1, d2

    def idx_wait(t, q):
        off = base + t * KCHUNK
        pltpu.make_async_copy(
            srcs_hbm.at[pl.ds(c * EPAD + off, KCHUNK)], srcs[q],
            isems[q]).wait()
        pltpu.make_async_copy(
            dst_hbm.at[pl.ds(off, KCHUNK)], dsts[q], isems[q]).wait()

    def g_issue(q):
        pltpu.async_copy(table_hbm.at[srcs[q]], rows[q], gsems[q])

    def g_wait(q):
        pltpu.make_async_copy(table_hbm.at[srcs[q]], rows[q], gsems[q]).wait()

    def s_issue(q):
        pltpu.async_copy(rows[q], agg_sh.at[dsts[q]], ssems[q], add=True)

    def s_wait(q):
        pltpu.make_async_copy(rows[q], agg_sh.at[dsts[q]], ssems[q]).wait()

    def step(t, q, first=False):
        q1 = (q + 1) % NSET
        q2 = (q + 2) % NSET
        if not first:
            s_wait(q2)
        idx_issue(t + 2, q2)
        idx_wait(t + 1, q1)
        g_issue(q1)
        g_wait(q)
        s_issue(q)

    idx_load(0, 0)
    g_issue(0)
    idx_issue(1, 1)
    step(0, 0, first=True)
    step(1, 1, first=True)
    step(2, 2)
    step(3, 3)

    def quad(p, carry):
        t = 4 * p
        for i in range(4):
            step(t + i, i)
        return carry

    lax.fori_loop(1, CHUNKS_PT // 4, quad, 0)
    g_wait(0)
    idx_wait(CHUNKS_PT + 1, (CHUNKS_PT + 1) % NSET)
    s_wait((CHUNKS_PT - 2) % NSET)
    s_wait((CHUNKS_PT - 1) % NSET)
    plsc.subcore_barrier()
    pltpu.sync_copy(agg_sh.at[pl.ds(s * ROWS_PT, ROWS_PT)],
                    out_hbm.at[pl.ds(c * NPAD + s * ROWS_PT, ROWS_PT)])


@functools.lru_cache(maxsize=1)
def _sc_agg_kernel():
    return pl.kernel(
        _sc_agg_body,
        out_type=jax.ShapeDtypeStruct((2 * NPAD, H), jnp.float32),
        mesh=plsc.VectorSubcoreMesh(
            core_axis_name="c", subcore_axis_name="s",
            num_cores=2, num_subcores=NTILES),
        scratch_types=(
            [pltpu.VMEM_SHARED((NPAD, H), jnp.float32)]
            + [pltpu.VMEM((KCHUNK,), jnp.int32) for _ in range(2 * NSET)]
            + [pltpu.VMEM((KCHUNK, H), jnp.float32) for _ in range(NSET)]
            + [pltpu.SemaphoreType.DMA for _ in range(3 * NSET)]
        ),
    )


def _agg_call(srcs, dstp, table, zeros_blk):
    out = _sc_agg_kernel()(srcs, dstp, table, zeros_blk)
    return out.reshape(2, NPAD, H)


def _prep_body(x_ref, xs0_ref, root_ref, cw_ref, cb_ref, aw_ref, ab_ref,
               tab_ref):
    i = pl.program_id(0)
    xb = x_ref[...]
    rows = lax.broadcasted_iota(jnp.int32, (BLK, G), 0) + i * BLK
    m = (rows == root_ref[...]).astype(jnp.float32)
    seed = jnp.minimum(jnp.sum(m, axis=1, keepdims=True), 1.0)
    h0 = jnp.concatenate(
        [xb, seed, jnp.zeros((BLK, H - NIN), jnp.float32)], axis=1)
    cond0 = jnp.dot(xs0_ref[...], cw_ref[...],
                    preferred_element_type=jnp.float32) + cb_ref[...]
    cfirst = jnp.dot(cond0, aw_ref[...],
                     preferred_element_type=jnp.float32) + ab_ref[...]
    tab_ref[0] = h0
    tab_ref[1] = h0 + cfirst


def _prep_call(x, xs0, root2d, cond_w, cond_b2d, aw, ab):
    full = lambda i: (0, 0)
    return pl.pallas_call(
        _prep_body,
        grid=(N // BLK,),
        in_specs=[
            pl.BlockSpec((BLK, POS), lambda i: (i, 0)),
            pl.BlockSpec((BLK, POS), lambda i: (i, 0)),
            pl.BlockSpec((1, G), full),
            pl.BlockSpec((POS, H), full),
            pl.BlockSpec((1, H), full),
            pl.BlockSpec((H, H), full),
            pl.BlockSpec((1, H), full),
        ],
        out_specs=pl.BlockSpec((2, BLK, H), lambda i: (0, i, 0)),
        out_shape=jax.ShapeDtypeStruct((2, N, H), jnp.float32),
    )(x, xs0, root2d, cond_w, cond_b2d, aw, ab)


def _layer_body(tab_ref, agg_ref, xsn_ref, wf_ref, bf_ref, wc_ref, bc_ref,
                zw_ref, zb_ref, cw_ref, cb_ref, acc_ref,
                tabn_ref, accn_ref):
    hf = jnp.maximum(
        jnp.dot(tab_ref[0] + agg_ref[0], wf_ref[...],
                preferred_element_type=jnp.float32) + bf_ref[...], 0.0)
    hc = jnp.maximum(
        jnp.dot(tab_ref[1] + agg_ref[1], wc_ref[...],
                preferred_element_type=jnp.float32) + bc_ref[...], 0.0)
    z = jnp.dot(hc, zw_ref[...], preferred_element_type=jnp.float32) \
        + zb_ref[...]
    hf_new = hf + RESIDUAL_SCALE * z
    condn = jnp.dot(xsn_ref[...], cw_ref[...],
                    preferred_element_type=jnp.float32) + cb_ref[...]
    tabn_ref[0] = hf_new
    tabn_ref[1] = hc + condn
    accn_ref[...] = acc_ref[...] + hf_new


def _layer_call(tab, agg, xsn, wf, bf, wc, bc, zw, zb, cond_w, cond_b2d, acc):
    full = lambda i: (0, 0)
    return pl.pallas_call(
        _layer_body,
        grid=(N // BLK,),
        in_specs=[
            pl.BlockSpec((2, BLK, H), lambda i: (0, i, 0)),
            pl.BlockSpec((2, BLK, H), lambda i: (0, i, 0)),
            pl.BlockSpec((BLK, POS), lambda i: (i, 0)),
            pl.BlockSpec((H, H), full),
            pl.BlockSpec((1, H), full),
            pl.BlockSpec((H, H), full),
            pl.BlockSpec((1, H), full),
            pl.BlockSpec((H, H), full),
            pl.BlockSpec((1, H), full),
            pl.BlockSpec((POS, H), full),
            pl.BlockSpec((1, H), full),
            pl.BlockSpec((BLK, H), lambda i: (i, 0)),
        ],
        out_specs=[
            pl.BlockSpec((2, BLK, H), lambda i: (0, i, 0)),
            pl.BlockSpec((BLK, H), lambda i: (i, 0)),
        ],
        out_shape=[
            jax.ShapeDtypeStruct((2, N, H), jnp.float32),
            jax.ShapeDtypeStruct((N, H), jnp.float32),
        ],
    )(tab, agg, xsn, wf, bf, wc, bc, zw, zb, cond_w, cond_b2d, acc)


def _pool_body(acc_ref, batch_ref, clsw_ref, clsb_ref, out_ref):
    oh = (lax.broadcasted_iota(jnp.int32, (G, N), 0)
          == batch_ref[...]).astype(jnp.float32)
    pooled = jnp.dot(oh, acc_ref[...], preferred_element_type=jnp.float32)
    cnt = jnp.sum(oh, axis=1, keepdims=True)
    pooled = pooled / jnp.maximum(cnt, 1.0)
    nrm = jnp.sqrt(jnp.sum(pooled * pooled, axis=1, keepdims=True))
    pooled = pooled / jnp.maximum(nrm, 1e-5)
    out_ref[...] = jnp.dot(pooled, clsw_ref[...],
                           preferred_element_type=jnp.float32) + clsb_ref[...]


def _pool_call(acc, batch2d, clsw_pad, clsb_pad):
    return pl.pallas_call(
        _pool_body,
        out_shape=jax.ShapeDtypeStruct((G, H), jnp.float32),
    )(acc, batch2d, clsw_pad, clsb_pad)


def kernel(x, x_sim_list, edge_index, batch, root_n_id, enc_w0, enc_b0,
           enc_w, enc_b, ctrl_w0, ctrl_b0, ctrl_w, ctrl_b, cond_w, cond_b,
           adapt_w, adapt_b, zero_w, zero_b, cls_w, cls_b):
    f32 = jnp.float32
    src = jnp.pad(edge_index[0], (0, EPAD - E))
    srcs = jnp.pad(jnp.concatenate([src, src + N]), (0, 2 * KCHUNK))
    dstp = jnp.pad(edge_index[1], (0, EPAD - E + 2 * KCHUNK),
                   constant_values=N)
    zeros_blk = jnp.zeros((KCHUNK, H), f32)

    wf0 = jnp.pad(enc_w0, ((0, H - NIN), (0, 0)))
    wc0 = jnp.pad(ctrl_w0, ((0, H - NIN), (0, 0)))
    aw = jnp.pad(adapt_w, ((0, 0), (0, H - NIN)))
    ab = jnp.pad(adapt_b, (0, H - NIN))[None]
    cond_b2d = cond_b[None]

    tab = _prep_call(x, x_sim_list[0], root_n_id[None].astype(jnp.int32),
                     cond_w, cond_b2d, aw, ab)
    acc = jnp.zeros((N, H), f32)
    for i in range(L):
        agg = _agg_call(srcs, dstp, tab.reshape(2 * N, H), zeros_blk)
        wf = wf0 if i == 0 else enc_w[i - 1]
        bf = (enc_b0 if i == 0 else enc_b[i - 1])[None]
        wc = wc0 if i == 0 else ctrl_w[i - 1]
        bc = (ctrl_b0 if i == 0 else ctrl_b[i - 1])[None]
        xsn = x_sim_list[(i + 1) % L]
        tab, acc = _layer_call(tab, agg, xsn, wf, bf, wc, bc,
                               zero_w[i], zero_b[i][None], cond_w, cond_b2d,
                               acc)
    out = _pool_call(acc, batch[None], jnp.pad(cls_w, ((0, 0), (0, H - C))),
                     jnp.pad(cls_b, (0, H - C))[None])
    return out[:, :C]
